# baseline (speedup 1.0000x reference)
import functools

import jax
import jax.numpy as jnp
from jax import lax
from jax.experimental import pallas as pl
from jax.experimental.pallas import tpu as pltpu
from jax.experimental.pallas import tpu_sc as plsc

_NC = 2
_NS = 16
_NW = _NC * _NS

_BT = 128
_L = 16


def _embedding_gather(w2, idx_h, par64):
    s_len, b_len = idx_h.shape
    n_bt = b_len // _BT
    assert n_bt == _NW

    mesh = plsc.VectorSubcoreMesh(core_axis_name="c", subcore_axis_name="s")

    @functools.partial(
        pl.kernel,
        out_type=jax.ShapeDtypeStruct((s_len, 8, n_bt, 8, _BT), jnp.float32),
        mesh=mesh,
        compiler_params=pltpu.CompilerParams(
            use_tc_tiling_on_sc=False, needs_layout_passes=False
        ),
        scratch_types=[
            pltpu.VMEM((s_len, _BT), jnp.int32),
            pltpu.VMEM((s_len, _BT), jnp.int32),
            pltpu.VMEM((2, _BT, _BT), jnp.float32),
            pltpu.VMEM((2, 8, 8, _BT), jnp.float32),
            [pltpu.SemaphoreType.DMA] * 2,
            [pltpu.SemaphoreType.DMA] * 2,
        ],
    )
    def body(
        table_hbm, idxh_hbm, par_hbm, out_hbm,
        idx_v, par_v, rows_v, rowst_v, gsems, wsems,
    ):
        wid = lax.axis_index("s") * _NC + lax.axis_index("c")

        pltpu.sync_copy(idxh_hbm.at[:, pl.ds(wid * _BT, _BT)], idx_v)
        pltpu.sync_copy(par_hbm.at[:, pl.ds(wid * _BT, _BT)], par_v)

        lane = lax.iota(jnp.int32, _L)
        bl_idx = [lane + _L * k for k in range(_BT // _L)]

        def fire_gather(it, bi):
            pltpu.async_copy(
                table_hbm.at[idx_v.at[it]], rows_v.at[bi], gsems[bi]
            )

        def drain_gather(bi):
            pltpu.make_async_copy(
                table_hbm.at[idx_v.at[0]], rows_v.at[bi], gsems[bi]
            ).wait()

        def fire_writeback(it, bi):
            pltpu.async_copy(rowst_v.at[bi], out_hbm.at[it, :, wid], wsems[bi])

        def drain_writeback(bi):
            pltpu.make_async_copy(
                rowst_v.at[bi], out_hbm.at[0, :, 0], wsems[bi]
            ).wait()

        def transpose(it, bi):
            cols = [par_v[it, pl.ds(k * _L, _L)] for k in range(_BT // _L)]

            def dt_step(dt, carry):
                for ds in range(8):
                    d = dt * 8 + ds
                    for k in range(_BT // _L):
                        vals = plsc.load_gather(
                            rows_v.at[bi], [bl_idx[k], cols[k] + d]
                        )
                        rowst_v[bi, dt, ds, pl.ds(k * _L, _L)] = vals
                return carry

            lax.fori_loop(0, 8, dt_step, 0)

        fire_gather(0, 0)
        fire_gather(1, 1)
        for bi in range(2):
            drain_gather(bi)
            transpose(bi, bi)
            fire_writeback(bi, bi)
            fire_gather(bi + 2, bi)

        def steady(i2, carry):
            for bi in range(2):
                it = 2 + 2 * i2 + bi
                drain_gather(bi)
                drain_writeback(bi)
                transpose(it, bi)
                fire_writeback(it, bi)
                fire_gather(it + 2, bi)
            return carry

        lax.fori_loop(0, (s_len - 4) // 2, steady, 0)

        for bi in range(2):
            it = s_len - 2 + bi
            drain_gather(bi)
            drain_writeback(bi)
            transpose(it, bi)
            fire_writeback(it, bi)
        for bi in range(2):
            drain_writeback(bi)

    return body(w2, idx_h, par64)


def kernel(input_ids, weight):
    b0, s = input_ids.shape
    v, d = weight.shape
    w2 = weight.reshape(v // 2, 2 * d)
    ids = input_ids.astype(jnp.int32)
    idx_h = (ids >> 1).T
    par64 = ((ids & 1) * d).T
    out5 = _embedding_gather(w2, idx_h, par64)
    return out5.transpose(2, 4, 0, 1, 3).reshape(b0, s, d)


# speedup vs baseline: 2.1183x; 2.1183x over previous
import functools

import jax
import jax.numpy as jnp
from jax import lax
from jax.experimental import pallas as pl
from jax.experimental.pallas import tpu as pltpu
from jax.experimental.pallas import tpu_sc as plsc

_NC = 2
_NS = 16
_NW = _NC * _NS

_BT = 128
_L = 16


def _embedding_gather(w2, idx_h, par64):
    s_len, b_len = idx_h.shape
    n_bt = b_len // _BT
    assert n_bt == _NW

    mesh = plsc.VectorSubcoreMesh(core_axis_name="c", subcore_axis_name="s")

    @functools.partial(
        pl.kernel,
        out_type=jax.ShapeDtypeStruct((s_len, 64 * b_len // _NW * _NW), jnp.float32),
        mesh=mesh,
        compiler_params=pltpu.CompilerParams(
            use_tc_tiling_on_sc=False, needs_layout_passes=False
        ),
        scratch_types=[
            pltpu.VMEM((s_len, _BT), jnp.int32),
            pltpu.VMEM((s_len, _BT), jnp.int32),
            pltpu.VMEM((2, _BT, _BT), jnp.float32),
            pltpu.VMEM((2, 64 * _BT), jnp.float32),
            [pltpu.SemaphoreType.DMA] * 2,
            [pltpu.SemaphoreType.DMA] * 2,
        ],
    )
    def body(
        table_hbm, idxh_hbm, par_hbm, out_hbm,
        idx_v, par_v, rows_v, rowst_v, gsems, wsems,
    ):
        wid = lax.axis_index("s") * _NC + lax.axis_index("c")

        pltpu.sync_copy(idxh_hbm.at[:, pl.ds(wid * _BT, _BT)], idx_v)
        pltpu.sync_copy(par_hbm.at[:, pl.ds(wid * _BT, _BT)], par_v)

        iota = lax.iota(jnp.int32, _L)
        rot = [(iota + j) & 15 for j in range(_L)]
        srot = [(((iota + j) & 15) << 7) + iota for j in range(_L)]

        def fire_gather(it, bi):
            pltpu.async_copy(
                table_hbm.at[idx_v.at[it]], rows_v.at[bi], gsems[bi]
            )

        def drain_gather(bi):
            pltpu.make_async_copy(
                table_hbm.at[idx_v.at[0]], rows_v.at[bi], gsems[bi]
            ).wait()

        def fire_writeback(it, bi):
            for dt in range(8):
                pltpu.async_copy(
                    rowst_v.at[bi, pl.ds(dt * 1024, 1024)],
                    out_hbm.at[it, pl.ds(dt * 32768 + wid * 1024, 1024)],
                    wsems[bi],
                )

        def drain_writeback(bi):
            for dt in range(8):
                pltpu.make_async_copy(
                    rowst_v.at[bi, pl.ds(dt * 1024, 1024)],
                    out_hbm.at[0, pl.ds(dt * 1024, 1024)],
                    wsems[bi],
                ).wait()

        def transpose(it, bi):
            @plsc.parallel_loop(0, 32)
            def g_step(g):
                bl0 = (g >> 2) * _L
                d0 = (g & 3) * _L
                parv = par_v[it, pl.ds(bl0, _L)]
                rowv = iota + bl0
                cbase = parv + d0
                sbase = d0 * _BT + bl0
                for j in range(_L):
                    vals = plsc.load_gather(
                        rows_v.at[bi], [rowv, cbase + rot[j]]
                    )
                    plsc.store_scatter(
                        rowst_v.at[bi], [srot[j] + sbase], vals
                    )

        fire_gather(0, 0)
        fire_gather(1, 1)
        for bi in range(2):
            drain_gather(bi)
            transpose(bi, bi)
            fire_writeback(bi, bi)
            fire_gather(bi + 2, bi)

        def steady(i2, carry):
            for bi in range(2):
                it = 2 + 2 * i2 + bi
                drain_gather(bi)
                drain_writeback(bi)
                transpose(it, bi)
                fire_writeback(it, bi)
                fire_gather(it + 2, bi)
            return carry

        lax.fori_loop(0, (s_len - 4) // 2, steady, 0)

        for bi in range(2):
            it = s_len - 2 + bi
            drain_gather(bi)
            drain_writeback(bi)
            transpose(it, bi)
            fire_writeback(it, bi)
        for bi in range(2):
            drain_writeback(bi)

    return body(w2, idx_h, par64)


def kernel(input_ids, weight):
    b0, s = input_ids.shape
    v, d = weight.shape
    w2 = weight.reshape(v // 2, 2 * d)
    ids = input_ids.astype(jnp.int32)
    idx_h = (ids >> 1).T
    par64 = ((ids & 1) * d).T
    out2 = _embedding_gather(w2, idx_h, par64)
    out5 = out2.reshape(s, 8, b0 // _BT, 8, _BT)
    return out5.transpose(2, 4, 0, 1, 3).reshape(b0, s, d)


# speedup vs baseline: 2.1219x; 1.0017x over previous
import functools

import jax
import jax.numpy as jnp
from jax import lax
from jax.experimental import pallas as pl
from jax.experimental.pallas import tpu as pltpu
from jax.experimental.pallas import tpu_sc as plsc

_NC = 2
_NS = 16
_NW = _NC * _NS

_BT = 128
_L = 16


def _embedding_gather(w2, idx_t):
    s_len, b_len = idx_t.shape
    n_bt = b_len // _BT
    assert n_bt == _NW

    mesh = plsc.VectorSubcoreMesh(core_axis_name="c", subcore_axis_name="s")

    @functools.partial(
        pl.kernel,
        out_type=jax.ShapeDtypeStruct((s_len, 64 * b_len // _NW * _NW), jnp.float32),
        mesh=mesh,
        compiler_params=pltpu.CompilerParams(
            use_tc_tiling_on_sc=False, needs_layout_passes=False
        ),
        scratch_types=[
            pltpu.VMEM((s_len, _BT), jnp.int32),
            pltpu.VMEM((s_len, _BT), jnp.int32),
            pltpu.VMEM((2, _BT, _BT), jnp.float32),
            pltpu.VMEM((2, 64 * _BT), jnp.float32),
            [pltpu.SemaphoreType.DMA] * 2,
            [pltpu.SemaphoreType.DMA] * 2,
        ],
    )
    def body(
        table_hbm, idx_hbm, out_hbm,
        idx_v, par_v, rows_v, rowst_v, gsems, wsems,
    ):
        wid = lax.axis_index("s") * _NC + lax.axis_index("c")

        pltpu.sync_copy(idx_hbm.at[:, pl.ds(wid * _BT, _BT)], idx_v)

        @plsc.parallel_loop(0, s_len)
        def prep(srow):
            for k in range(_BT // _L):
                v = idx_v[srow, pl.ds(k * _L, _L)]
                par_v[srow, pl.ds(k * _L, _L)] = (v & 1) << 6
                idx_v[srow, pl.ds(k * _L, _L)] = v >> 1

        iota = lax.iota(jnp.int32, _L)
        rot = [(iota + j) & 15 for j in range(_L)]
        srot = [(((iota + j) & 15) << 7) + iota for j in range(_L)]

        def fire_gather(it, bi):
            pltpu.async_copy(
                table_hbm.at[idx_v.at[it]], rows_v.at[bi], gsems[bi]
            )

        def drain_gather(bi):
            pltpu.make_async_copy(
                table_hbm.at[idx_v.at[0]], rows_v.at[bi], gsems[bi]
            ).wait()

        def fire_writeback(it, bi):
            for dt in range(8):
                pltpu.async_copy(
                    rowst_v.at[bi, pl.ds(dt * 1024, 1024)],
                    out_hbm.at[it, pl.ds(dt * 32768 + wid * 1024, 1024)],
                    wsems[bi],
                )

        def drain_writeback(bi):
            for dt in range(8):
                pltpu.make_async_copy(
                    rowst_v.at[bi, pl.ds(dt * 1024, 1024)],
                    out_hbm.at[0, pl.ds(dt * 1024, 1024)],
                    wsems[bi],
                ).wait()

        def transpose(it, bi):
            @plsc.parallel_loop(0, 32)
            def g_step(g):
                bl0 = (g >> 2) * _L
                d0 = (g & 3) * _L
                parv = par_v[it, pl.ds(bl0, _L)]
                rowv = iota + bl0
                cbase = parv + d0
                sbase = d0 * _BT + bl0
                for j in range(_L):
                    vals = plsc.load_gather(
                        rows_v.at[bi], [rowv, cbase + rot[j]]
                    )
                    plsc.store_scatter(
                        rowst_v.at[bi], [srot[j] + sbase], vals
                    )

        fire_gather(0, 0)
        fire_gather(1, 1)
        for bi in range(2):
            drain_gather(bi)
            transpose(bi, bi)
            fire_writeback(bi, bi)
            fire_gather(bi + 2, bi)

        def steady(i2, carry):
            for bi in range(2):
                it = 2 + 2 * i2 + bi
                drain_gather(bi)
                drain_writeback(bi)
                transpose(it, bi)
                fire_writeback(it, bi)
                fire_gather(it + 2, bi)
            return carry

        lax.fori_loop(0, (s_len - 4) // 2, steady, 0)

        for bi in range(2):
            it = s_len - 2 + bi
            drain_gather(bi)
            drain_writeback(bi)
            transpose(it, bi)
            fire_writeback(it, bi)
        for bi in range(2):
            drain_writeback(bi)

    return body(w2, idx_t)


def kernel(input_ids, weight):
    b0, s = input_ids.shape
    v, d = weight.shape
    w2 = weight.reshape(v // 2, 2 * d)
    idx_t = input_ids.astype(jnp.int32).T
    out2 = _embedding_gather(w2, idx_t)
    out5 = out2.reshape(s, 8, b0 // _BT, 8, _BT)
    return out5.transpose(2, 4, 0, 1, 3).reshape(b0, s, d)


# speedup vs baseline: 2.3136x; 1.0903x over previous
import functools

import jax
import jax.numpy as jnp
from jax import lax
from jax.experimental import pallas as pl
from jax.experimental.pallas import tpu as pltpu
from jax.experimental.pallas import tpu_sc as plsc

_NC = 2
_NS = 16
_NW = _NC * _NS

_BT = 128
_L = 16


def _embedding_gather(w, idx_t):
    s_len, b_len = idx_t.shape
    v_len, d_len = w.shape
    n_bt = b_len // _BT
    assert n_bt == _NW and d_len == 64

    mesh = plsc.VectorSubcoreMesh(core_axis_name="c", subcore_axis_name="s")

    @functools.partial(
        pl.kernel,
        out_type=jax.ShapeDtypeStruct((s_len, 64 * b_len), jnp.float32),
        mesh=mesh,
        compiler_params=pltpu.CompilerParams(
            use_tc_tiling_on_sc=False, needs_layout_passes=False
        ),
        scratch_types=[
            pltpu.VMEM((s_len, _BT), jnp.int32),
            pltpu.VMEM((2, _BT, 64), jnp.float32),
            pltpu.VMEM((2, 64 * _BT), jnp.float32),
            [pltpu.SemaphoreType.DMA] * 2,
            [pltpu.SemaphoreType.DMA] * 2,
        ],
    )
    def body(
        table_hbm, idx_hbm, out_hbm,
        idx_v, rows_v, rowst_v, gsems, wsems,
    ):
        wid = lax.axis_index("s") * _NC + lax.axis_index("c")

        pltpu.sync_copy(idx_hbm.at[:, pl.ds(wid * _BT, _BT)], idx_v)

        iota = lax.iota(jnp.int32, _L)
        rot = [(iota + j) & 15 for j in range(_L)]
        srot = [(((iota + j) & 15) << 7) + iota for j in range(_L)]

        def fire_gather(it, bi):
            pltpu.async_copy(
                table_hbm.at[idx_v.at[it]], rows_v.at[bi], gsems[bi]
            )

        def drain_gather(bi):
            pltpu.make_async_copy(
                table_hbm.at[idx_v.at[0]], rows_v.at[bi], gsems[bi]
            ).wait()

        def fire_writeback(it, bi):
            for dt in range(8):
                pltpu.async_copy(
                    rowst_v.at[bi, pl.ds(dt * 1024, 1024)],
                    out_hbm.at[it, pl.ds(dt * 32768 + wid * 1024, 1024)],
                    wsems[bi],
                )

        def drain_writeback(bi):
            for dt in range(8):
                pltpu.make_async_copy(
                    rowst_v.at[bi, pl.ds(dt * 1024, 1024)],
                    out_hbm.at[0, pl.ds(dt * 1024, 1024)],
                    wsems[bi],
                ).wait()

        def transpose(it, bi):
            @plsc.parallel_loop(0, 32)
            def g_step(g):
                bl0 = (g >> 2) * _L
                d0 = (g & 3) * _L
                rowv = iota + bl0
                sbase = d0 * _BT + bl0
                for j in range(_L):
                    vals = plsc.load_gather(
                        rows_v.at[bi], [rowv, rot[j] + d0]
                    )
                    plsc.store_scatter(
                        rowst_v.at[bi], [srot[j] + sbase], vals
                    )

        fire_gather(0, 0)
        fire_gather(1, 1)
        for bi in range(2):
            drain_gather(bi)
            transpose(bi, bi)
            fire_writeback(bi, bi)
            fire_gather(bi + 2, bi)

        def steady(i2, carry):
            for bi in range(2):
                it = 2 + 2 * i2 + bi
                drain_gather(bi)
                drain_writeback(bi)
                transpose(it, bi)
                fire_writeback(it, bi)
                fire_gather(it + 2, bi)
            return carry

        lax.fori_loop(0, (s_len - 4) // 2, steady, 0)

        for bi in range(2):
            it = s_len - 2 + bi
            drain_gather(bi)
            drain_writeback(bi)
            transpose(it, bi)
            fire_writeback(it, bi)
        for bi in range(2):
            drain_writeback(bi)

    return body(w, idx_t)


def kernel(input_ids, weight):
    b0, s = input_ids.shape
    v, d = weight.shape
    idx_t = input_ids.astype(jnp.int32).T
    out2 = _embedding_gather(weight, idx_t)
    out5 = out2.reshape(s, 8, b0 // _BT, 8, _BT)
    return out5.transpose(2, 4, 0, 1, 3).reshape(b0, s, d)


# speedup vs baseline: 4.2563x; 1.8397x over previous
import functools

import jax
import jax.numpy as jnp
from jax import lax
from jax.experimental import pallas as pl
from jax.experimental.pallas import tpu as pltpu
from jax.experimental.pallas import tpu_sc as plsc

_NC = 2
_NS = 16
_NW = _NC * _NS

_BT = 128
_L = 16


def _format_weight(wT, tail):
    d_len, v_len = wT.shape
    n_full = v_len // _BT
    n_main = (n_full // _NW) * _NW
    k_per_w = n_main // _NW
    n_rest = n_full - n_main

    mesh = plsc.VectorSubcoreMesh(core_axis_name="c", subcore_axis_name="s")

    @functools.partial(
        pl.kernel,
        out_type=jax.ShapeDtypeStruct((v_len // 2, 2 * d_len), jnp.float32),
        mesh=mesh,
        compiler_params=pltpu.CompilerParams(
            use_tc_tiling_on_sc=True, needs_layout_passes=False
        ),
        scratch_types=[
            pltpu.VMEM((2, 64, _BT), jnp.float32),
            pltpu.VMEM((2, 64, _BT), jnp.float32),
            [pltpu.SemaphoreType.DMA] * 2,
            [pltpu.SemaphoreType.DMA] * 2,
        ],
    )
    def body(wt_hbm, tail_hbm, out_hbm, cols_v, rowsp_v, gsems, wsems):
        wid = lax.axis_index("s") * _NC + lax.axis_index("c")

        iota = lax.iota(jnp.int32, _L)

        def c0_of(k):
            return (wid + _NW * k) * _BT

        def fire_in(k, bi):
            pltpu.async_copy(
                wt_hbm.at[:, pl.ds(c0_of(k), _BT)], cols_v.at[bi], gsems[bi]
            )

        def drain_in(bi):
            pltpu.make_async_copy(
                wt_hbm.at[:, pl.ds(0, _BT)], cols_v.at[bi], gsems[bi]
            ).wait()

        def fire_out(k, bi):
            pltpu.async_copy(
                rowsp_v.at[bi],
                out_hbm.at[pl.ds(pl.multiple_of(c0_of(k) >> 1, 64), 64)],
                wsems[bi],
            )

        def drain_out(bi):
            pltpu.make_async_copy(
                rowsp_v.at[bi], out_hbm.at[pl.ds(0, 64)], wsems[bi]
            ).wait()

        def repack(bi):
            @plsc.parallel_loop(0, 32)
            def g_step(g):
                r0 = (g & 3) * _L
                cg = g >> 2
                h = cg >> 2
                d0 = (cg & 3) * _L
                for j in range(_L):
                    dvec = ((iota + j) & 15) + d0
                    vals = plsc.load_gather(
                        cols_v.at[bi],
                        [dvec, (iota << 1) + (2 * r0 + h)],
                    )
                    plsc.store_scatter(
                        rowsp_v.at[bi],
                        [iota + r0, dvec + h * 64],
                        vals,
                    )

        fire_in(0, 0)
        fire_in(1, 1)
        for bi in range(2):
            drain_in(bi)
            repack(bi)
            fire_out(bi, bi)
            fire_in(bi + 2, bi)

        def steady(i2, carry):
            for bi in range(2):
                k = 2 + 2 * i2 + bi
                drain_in(bi)
                drain_out(bi)
                repack(bi)
                fire_out(k, bi)
                fire_in(k + 2, bi)
            return carry

        lax.fori_loop(0, (k_per_w - 4) // 2, steady, 0)

        for bi in range(2):
            k = k_per_w - 2 + bi
            drain_in(bi)
            drain_out(bi)
            repack(bi)
            fire_out(k, bi)
        for bi in range(2):
            drain_out(bi)

        @pl.when(wid < n_rest)
        def _tail():
            c0 = pl.multiple_of((n_main + wid) * _BT, _BT)
            pltpu.sync_copy(wt_hbm.at[:, pl.ds(c0, _BT)], cols_v.at[0])
            repack(0)
            pltpu.sync_copy(
                rowsp_v.at[0],
                out_hbm.at[pl.ds(pl.multiple_of(c0 >> 1, 64), 64)],
            )

        @pl.when(wid == n_rest)
        def _tail2():
            pltpu.sync_copy(tail_hbm, rowsp_v.at[0].at[pl.ds(0, 32)])
            pltpu.sync_copy(
                rowsp_v.at[0].at[pl.ds(0, 32)],
                out_hbm.at[pl.ds(v_len // 2 - 32, 32)],
            )

    return body(wT, tail)


def _embedding_gather(w, idx_t):
    s_len, b_len = idx_t.shape
    v_len, d_len = w.shape
    n_bt = b_len // _BT
    assert n_bt == _NW and d_len == 64

    mesh = plsc.VectorSubcoreMesh(core_axis_name="c", subcore_axis_name="s")

    @functools.partial(
        pl.kernel,
        out_type=jax.ShapeDtypeStruct((s_len, 64 * b_len), jnp.float32),
        mesh=mesh,
        compiler_params=pltpu.CompilerParams(
            use_tc_tiling_on_sc=False, needs_layout_passes=False
        ),
        scratch_types=[
            pltpu.VMEM((s_len, _BT), jnp.int32),
            pltpu.VMEM((2, _BT, 64), jnp.float32),
            pltpu.VMEM((2, 64 * _BT), jnp.float32),
            [pltpu.SemaphoreType.DMA] * 2,
            [pltpu.SemaphoreType.DMA] * 2,
        ],
    )
    def body(
        table_hbm, idx_hbm, out_hbm,
        idx_v, rows_v, rowst_v, gsems, wsems,
    ):
        wid = lax.axis_index("s") * _NC + lax.axis_index("c")

        pltpu.sync_copy(idx_hbm.at[:, pl.ds(wid * _BT, _BT)], idx_v)

        iota = lax.iota(jnp.int32, _L)
        rot = [(iota + j) & 15 for j in range(_L)]
        srot = [(((iota + j) & 15) << 7) + iota for j in range(_L)]

        def fire_gather(it, bi):
            pltpu.async_copy(
                table_hbm.at[idx_v.at[it]], rows_v.at[bi], gsems[bi]
            )

        def drain_gather(bi):
            pltpu.make_async_copy(
                table_hbm.at[idx_v.at[0]], rows_v.at[bi], gsems[bi]
            ).wait()

        def fire_writeback(it, bi):
            for dt in range(8):
                pltpu.async_copy(
                    rowst_v.at[bi, pl.ds(dt * 1024, 1024)],
                    out_hbm.at[it, pl.ds(dt * 32768 + wid * 1024, 1024)],
                    wsems[bi],
                )

        def drain_writeback(bi):
            for dt in range(8):
                pltpu.make_async_copy(
                    rowst_v.at[bi, pl.ds(dt * 1024, 1024)],
                    out_hbm.at[0, pl.ds(dt * 1024, 1024)],
                    wsems[bi],
                ).wait()

        def transpose(it, bi):
            @plsc.parallel_loop(0, 32)
            def g_step(g):
                bl0 = (g >> 2) * _L
                d0 = (g & 3) * _L
                rowv = iota + bl0
                sbase = d0 * _BT + bl0
                for j in range(_L):
                    vals = plsc.load_gather(
                        rows_v.at[bi], [rowv, rot[j] + d0]
                    )
                    plsc.store_scatter(
                        rowst_v.at[bi], [srot[j] + sbase], vals
                    )

        fire_gather(0, 0)
        fire_gather(1, 1)
        for bi in range(2):
            drain_gather(bi)
            transpose(bi, bi)
            fire_writeback(bi, bi)
            fire_gather(bi + 2, bi)

        def steady(i2, carry):
            for bi in range(2):
                it = 2 + 2 * i2 + bi
                drain_gather(bi)
                drain_writeback(bi)
                transpose(it, bi)
                fire_writeback(it, bi)
                fire_gather(it + 2, bi)
            return carry

        lax.fori_loop(0, (s_len - 4) // 2, steady, 0)

        for bi in range(2):
            it = s_len - 2 + bi
            drain_gather(bi)
            drain_writeback(bi)
            transpose(it, bi)
            fire_writeback(it, bi)
        for bi in range(2):
            drain_writeback(bi)

    return body(w, idx_t)


def kernel(input_ids, weight):
    b0, s = input_ids.shape
    v, d = weight.shape
    tail = weight[v - 64 :].reshape(32, 2 * d)
    w2p = _format_weight(weight.T, tail)
    w_lin = w2p.reshape(v, d)
    idx_t = input_ids.astype(jnp.int32).T
    out2 = _embedding_gather(w_lin, idx_t)
    out5 = out2.reshape(s, 8, b0 // _BT, 8, _BT)
    return out5.transpose(2, 4, 0, 1, 3).reshape(b0, s, d)
